# baseline (speedup 1.0000x reference)
import jax
import jax.numpy as jnp
from jax.experimental import pallas as pl
from jax.experimental.pallas import tpu as pltpu

_N_TOKENS = 32768
_DIM = 768
_CHUNK = 8192
_N_CHUNKS = _N_TOKENS // _CHUNK
_NBUF = 2


def _ring_kernel(x_hbm, o_hbm, buf, insem, outsem):
    def in_dma(chunk, slot):
        return pltpu.make_async_copy(
            x_hbm.at[pl.ds(chunk * _CHUNK, _CHUNK), :], buf.at[slot], insem.at[slot])

    def out_dma(chunk, slot):
        return pltpu.make_async_copy(
            buf.at[slot], o_hbm.at[pl.ds(chunk * _CHUNK, _CHUNK), :], outsem.at[slot])

    for i in range(_NBUF):
        in_dma(i, i).start()
    for i in range(_N_CHUNKS):
        slot = i % _NBUF
        in_dma(i, slot).wait()
        out_dma(i, slot).start()
        j = i + _NBUF
        if j < _N_CHUNKS:
            out_dma(i, slot).wait()
            in_dma(j, slot).start()
    for i in range(_N_CHUNKS - _NBUF, _N_CHUNKS):
        out_dma(i, i % _NBUF).wait()


def kernel(x, W, b):
    del W, b
    return pl.pallas_call(
        _ring_kernel,
        in_specs=[pl.BlockSpec(memory_space=pl.ANY)],
        out_specs=pl.BlockSpec(memory_space=pl.ANY),
        out_shape=jax.ShapeDtypeStruct((_N_TOKENS, _DIM), jnp.float32),
        scratch_shapes=[
            pltpu.VMEM((_NBUF, _CHUNK, _DIM), jnp.float32),
            pltpu.SemaphoreType.DMA((_NBUF,)),
            pltpu.SemaphoreType.DMA((_NBUF,)),
        ],
    )(x)


# speedup vs baseline: 1.0155x; 1.0155x over previous
import jax
import jax.numpy as jnp
from jax.experimental import pallas as pl
from jax.experimental.pallas import tpu as pltpu

_N_TOKENS = 32768
_DIM = 768
_CHUNK = 2048
_N_CHUNKS = _N_TOKENS // _CHUNK
_NBUF = 8


def _ring_kernel(x_hbm, o_hbm, buf, insem, outsem):
    def in_dma(chunk, slot):
        return pltpu.make_async_copy(
            x_hbm.at[pl.ds(chunk * _CHUNK, _CHUNK), :], buf.at[slot], insem.at[slot])

    def out_dma(chunk, slot):
        return pltpu.make_async_copy(
            buf.at[slot], o_hbm.at[pl.ds(chunk * _CHUNK, _CHUNK), :], outsem.at[slot])

    for i in range(_NBUF):
        in_dma(i, i).start()
    for i in range(_N_CHUNKS):
        slot = i % _NBUF
        in_dma(i, slot).wait()
        out_dma(i, slot).start()
        j = i + _NBUF
        if j < _N_CHUNKS:
            out_dma(i, slot).wait()
            in_dma(j, slot).start()
    for i in range(_N_CHUNKS - _NBUF, _N_CHUNKS):
        out_dma(i, i % _NBUF).wait()


def kernel(x, W, b):
    del W, b
    return pl.pallas_call(
        _ring_kernel,
        in_specs=[pl.BlockSpec(memory_space=pl.ANY)],
        out_specs=pl.BlockSpec(memory_space=pl.ANY),
        out_shape=jax.ShapeDtypeStruct((_N_TOKENS, _DIM), jnp.float32),
        scratch_shapes=[
            pltpu.VMEM((_NBUF, _CHUNK, _DIM), jnp.float32),
            pltpu.SemaphoreType.DMA((_NBUF,)),
            pltpu.SemaphoreType.DMA((_NBUF,)),
        ],
    )(x)
